# baseline (speedup 1.0000x reference)
import functools

import jax
import jax.numpy as jnp
from jax import lax
from jax.experimental import pallas as pl
from jax.experimental.pallas import tpu as pltpu
from jax.experimental.pallas import tpu_sc as plsc

_VOCAB = 128
_D = 32
_ROWS = 4096
_COLS = 200
_B = _ROWS * _COLS
_NC = 2
_NS = 16
_NW = _NC * _NS
_BPW = _B // _NW
_K = 1024
_NCH = _BPW // _K
_NG = _K // 16
_TABW = _VOCAB * _D


def _emb_body(ids_hbm, table_hbm, out_hbm, table_v, idx_v, rows_v, sem_o):
    wid = lax.axis_index("s") * _NC + lax.axis_index("c")
    in_base = wid * _BPW
    out_base = wid * _BPW * _D

    pltpu.sync_copy(table_hbm, table_v)

    lane = lax.iota(jnp.int32, 16)
    lane32 = lane * _D

    def gather_chunk(slot, c0):
        rbase0 = slot * (_K * _D)

        def group(g, c):
            ids16 = idx_v[pl.ds(slot * _K + g * 16, 16)]
            src0 = ids16 * _D
            dst0 = rbase0 + g * (16 * _D) + lane32
            for j in range(_D):
                vals = plsc.load_gather(table_v, [src0 + j])
                plsc.store_scatter(rows_v, [dst0 + j], vals)
            return c

        lax.fori_loop(0, _NG, group, c0)

    def writeout(i, slot):
        pltpu.async_copy(
            rows_v.at[pl.ds(slot * (_K * _D), _K * _D)],
            out_hbm.at[pl.ds(out_base + i * (_K * _D), _K * _D)],
            sem_o,
        )

    def drain_writeout():
        pltpu.make_async_copy(
            rows_v.at[pl.ds(0, _K * _D)],
            out_hbm.at[pl.ds(out_base, _K * _D)],
            sem_o,
        ).wait()

    pltpu.sync_copy(ids_hbm.at[pl.ds(in_base, _K)], idx_v.at[pl.ds(0, _K)])
    gather_chunk(0, 0)

    def body(i, c):
        slot = i % 2
        nslot = (i + 1) % 2
        writeout(i, slot)
        pltpu.sync_copy(
            ids_hbm.at[pl.ds(in_base + (i + 1) * _K, _K)],
            idx_v.at[pl.ds(nslot * _K, _K)],
        )
        gather_chunk(nslot, c)
        drain_writeout()
        return c

    lax.fori_loop(0, _NCH - 1, body, 0)

    writeout(_NCH - 1, (_NCH - 1) % 2)
    drain_writeout()


_emb = functools.partial(
    pl.kernel,
    mesh=plsc.VectorSubcoreMesh(core_axis_name="c", subcore_axis_name="s"),
    out_type=jax.ShapeDtypeStruct((_B * _D,), jnp.float32),
    scratch_types=[
        pltpu.VMEM((_TABW,), jnp.float32),
        pltpu.VMEM((2 * _K,), jnp.int32),
        pltpu.VMEM((2 * _K * _D,), jnp.float32),
        pltpu.SemaphoreType.DMA,
    ],
    compiler_params=pltpu.CompilerParams(
        use_tc_tiling_on_sc=False, needs_layout_passes=False
    ),
)(_emb_body)


@jax.jit
def kernel(input_ids, embed_weight):
    ids = input_ids.astype(jnp.int32).reshape(-1)
    out = _emb(ids, embed_weight.reshape(-1))
    return out.reshape(_ROWS, _COLS, _D)


# speedup vs baseline: 1.2122x; 1.2122x over previous
import functools

import jax
import jax.numpy as jnp
from jax import lax
from jax.experimental import pallas as pl
from jax.experimental.pallas import tpu as pltpu
from jax.experimental.pallas import tpu_sc as plsc

_VOCAB = 128
_D = 32
_ROWS = 4096
_COLS = 200
_B = _ROWS * _COLS
_NC = 2
_NS = 16
_NW = _NC * _NS
_BPW = _B // _NW
_K = 1024
_NCH = _BPW // _K
_NG = _K // 16
_TABW = _VOCAB * _D


def _emb_body(ids_hbm, table_hbm, out_hbm, table_v, idx_v, rows_v, sem_o):
    wid = lax.axis_index("s") * _NC + lax.axis_index("c")
    in_base = wid * _BPW
    out_base = wid * _BPW * _D

    pltpu.sync_copy(table_hbm, table_v)

    lane = lax.iota(jnp.int32, 16)
    lane32 = lane * _D

    def gather_chunk(slot):
        rbase0 = slot * (_K * _D)

        @plsc.parallel_loop(0, _NG, 1, unroll=2)
        def group(g):
            ids16 = idx_v[pl.ds(slot * _K + g * 16, 16)]
            src0 = ids16 * _D
            dst0 = rbase0 + g * (16 * _D) + lane32
            vals = [plsc.load_gather(table_v, [src0 + j]) for j in range(_D)]
            for j in range(_D):
                plsc.store_scatter(rows_v, [dst0 + j], vals[j])

    def writeout(i, slot):
        pltpu.async_copy(
            rows_v.at[pl.ds(slot * (_K * _D), _K * _D)],
            out_hbm.at[pl.ds(out_base + i * (_K * _D), _K * _D)],
            sem_o,
        )

    def drain_writeout():
        pltpu.make_async_copy(
            rows_v.at[pl.ds(0, _K * _D)],
            out_hbm.at[pl.ds(out_base, _K * _D)],
            sem_o,
        ).wait()

    pltpu.sync_copy(ids_hbm.at[pl.ds(in_base, _K)], idx_v.at[pl.ds(0, _K)])
    gather_chunk(0)

    def body(i, c):
        slot = i % 2
        nslot = (i + 1) % 2
        writeout(i, slot)
        pltpu.sync_copy(
            ids_hbm.at[pl.ds(in_base + (i + 1) * _K, _K)],
            idx_v.at[pl.ds(nslot * _K, _K)],
        )
        gather_chunk(nslot)
        drain_writeout()
        return c

    lax.fori_loop(0, _NCH - 1, body, 0)

    writeout(_NCH - 1, (_NCH - 1) % 2)
    drain_writeout()


_emb = functools.partial(
    pl.kernel,
    mesh=plsc.VectorSubcoreMesh(core_axis_name="c", subcore_axis_name="s"),
    out_type=jax.ShapeDtypeStruct((_B * _D,), jnp.float32),
    scratch_types=[
        pltpu.VMEM((_TABW,), jnp.float32),
        pltpu.VMEM((2 * _K,), jnp.int32),
        pltpu.VMEM((2 * _K * _D,), jnp.float32),
        pltpu.SemaphoreType.DMA,
    ],
    compiler_params=pltpu.CompilerParams(
        use_tc_tiling_on_sc=False, needs_layout_passes=False
    ),
)(_emb_body)


@jax.jit
def kernel(input_ids, embed_weight):
    ids = input_ids.astype(jnp.int32).reshape(-1)
    out = _emb(ids, embed_weight.reshape(-1))
    return out.reshape(_ROWS, _COLS, _D)


# speedup vs baseline: 10.8098x; 8.9178x over previous
import functools

import jax
import jax.numpy as jnp
from jax import lax
from jax.experimental import pallas as pl
from jax.experimental.pallas import tpu as pltpu
from jax.experimental.pallas import tpu_sc as plsc

_VOCAB = 128
_D = 32
_ROWS = 4096
_COLS = 200
_NC = 2
_NS = 16
_NW = _NC * _NS
_IW = _ROWS // _NW
_TABW = _VOCAB * _D


def _emb_body(ids_hbm, table_hbm, out_hbm, table_v, idx_v, blk0, blk1, sem_o):
    wid = lax.axis_index("s") * _NC + lax.axis_index("c")
    i0 = wid * _IW

    pltpu.sync_copy(table_hbm, table_v)
    pltpu.sync_copy(ids_hbm.at[pl.ds(0, _COLS), pl.ds(i0, _IW)], idx_v)

    lane = lax.iota(jnp.int32, 16)

    def gather_j(j, blk):
        @plsc.parallel_loop(0, _IW // 16, 1, unroll=1)
        def g_loop(g):
            ids16 = idx_v[j, pl.ds(g * 16, 16)]
            src0 = ids16 * _D
            div = lane + g * 16
            for tb in range(0, _D, 8):
                cols = [(lane + (tb + t)) & (_D - 1) for t in range(8)]
                vals = [
                    plsc.load_gather(table_v, [src0 + cols[t]])
                    for t in range(8)
                ]
                for t in range(8):
                    plsc.store_scatter(blk, [cols[t], div], vals[t])

    def fire(j, blk):
        pltpu.async_copy(
            blk, out_hbm.at[j, pl.ds(0, _D), pl.ds(i0, _IW)], sem_o
        )

    def drain():
        pltpu.make_async_copy(
            blk0, out_hbm.at[0, pl.ds(0, _D), pl.ds(i0, _IW)], sem_o
        ).wait()

    gather_j(0, blk0)
    fire(0, blk0)
    gather_j(1, blk1)
    fire(1, blk1)

    def body(it, c):
        j0 = 2 + 2 * it
        drain()
        gather_j(j0, blk0)
        fire(j0, blk0)
        drain()
        gather_j(j0 + 1, blk1)
        fire(j0 + 1, blk1)
        return c

    lax.fori_loop(0, (_COLS - 2) // 2, body, 0)
    drain()
    drain()


_emb = functools.partial(
    pl.kernel,
    mesh=plsc.VectorSubcoreMesh(core_axis_name="c", subcore_axis_name="s"),
    out_type=jax.ShapeDtypeStruct((_COLS, _D, _ROWS), jnp.float32),
    scratch_types=[
        pltpu.VMEM((_TABW,), jnp.float32),
        pltpu.VMEM((_COLS, _IW), jnp.int32),
        pltpu.VMEM((_D, _IW), jnp.float32),
        pltpu.VMEM((_D, _IW), jnp.float32),
        pltpu.SemaphoreType.DMA,
    ],
    compiler_params=pltpu.CompilerParams(
        use_tc_tiling_on_sc=True, needs_layout_passes=False
    ),
)(_emb_body)


@jax.jit
def kernel(input_ids, embed_weight):
    ids_t = input_ids.T.astype(jnp.int32)
    out_t = _emb(ids_t, embed_weight.reshape(-1))
    return out_t.transpose(2, 0, 1)
